# baseline (speedup 1.0000x reference)
import functools

import jax
import jax.numpy as jnp
from jax import lax
from jax.experimental import pallas as pl
from jax.experimental.pallas import tpu as pltpu
from jax.experimental.pallas import tpu_sc as plsc

N_NODES = 100000
N_NODES_PAD = 102400
N_EDGES = 6400000
NUM_WORKERS = 32
ROWS = N_EDGES // 128
CHUNK_ROWS = 25
CHUNK = CHUNK_ROWS * 128
N_CHUNKS = ROWS // CHUNK_ROWS
MAX_PAIRS = (N_CHUNKS // NUM_WORKERS + 2) // 2


def _sigmoid_body(x_ref, o_ref):
    o_ref[...] = 1.0 / (1.0 + jnp.exp(-x_ref[...]))


def _node_sigmoid(logits):
    x = jnp.pad(logits.reshape(-1), (0, N_NODES_PAD - N_NODES))
    y = pl.pallas_call(
        _sigmoid_body,
        out_shape=jax.ShapeDtypeStruct((N_NODES_PAD // 128, 128), jnp.float32),
    )(x.reshape(N_NODES_PAD // 128, 128))
    return y.reshape(-1)


_mesh = plsc.VectorSubcoreMesh(core_axis_name="c", subcore_axis_name="s")


@functools.partial(
    pl.kernel,
    mesh=_mesh,
    compiler_params=pltpu.CompilerParams(
        use_tc_tiling_on_sc=False, needs_layout_passes=False
    ),
    out_type=jax.ShapeDtypeStruct((ROWS, 128), jnp.float32),
    scratch_types=[
        pltpu.VMEM((N_NODES_PAD,), jnp.float32),
        pltpu.VMEM((2, CHUNK), jnp.int32),
        pltpu.VMEM((2, CHUNK), jnp.int32),
        pltpu.VMEM((2, CHUNK_ROWS, 128), jnp.float32),
        pltpu.SemaphoreType.DMA,
        pltpu.SemaphoreType.DMA,
        pltpu.SemaphoreType.DMA,
        pltpu.SemaphoreType.DMA,
    ],
)
def _edge_attn_sc(
    table_hbm, ei_hbm, out_hbm,
    table_v, src_v, dst_v, out_v,
    sin0, sin1, sout0, sout1,
):
    wid = lax.axis_index("s") * 2 + lax.axis_index("c")
    n_my = (N_CHUNKS - wid + NUM_WORKERS - 1) // NUM_WORKERS
    sin = (sin0, sin1)
    sout = (sout0, sout1)

    def start_in(k, b):
        off = (wid + k * NUM_WORKERS) * CHUNK
        pltpu.async_copy(ei_hbm.at[0, pl.ds(off, CHUNK)], src_v.at[b], sin[b])
        pltpu.async_copy(ei_hbm.at[1, pl.ds(off, CHUNK)], dst_v.at[b], sin[b])

    def wait_in(b):
        pltpu.make_async_copy(
            ei_hbm.at[0, pl.ds(0, CHUNK)], src_v.at[b], sin[b]
        ).wait()
        pltpu.make_async_copy(
            ei_hbm.at[1, pl.ds(0, CHUNK)], dst_v.at[b], sin[b]
        ).wait()

    def start_out(k, b):
        row0 = (wid + k * NUM_WORKERS) * CHUNK_ROWS
        pltpu.async_copy(
            out_v.at[b], out_hbm.at[pl.ds(row0, CHUNK_ROWS)], sout[b]
        )

    def wait_out(b):
        pltpu.make_async_copy(
            out_v.at[b], out_hbm.at[pl.ds(0, CHUNK_ROWS)], sout[b]
        ).wait()

    start_in(0, 0)
    start_in(1, 1)
    pltpu.sync_copy(table_hbm, table_v)

    lane = lax.iota(jnp.int32, 16)

    def pair_body(p, carry):
        for b in range(2):
            k = p * 2 + b

            @pl.when(k < n_my)
            def _():
                wait_in(b)

                @pl.when(p > 0)
                def _():
                    wait_out(b)

                sv = src_v.at[b]
                dv = dst_v.at[b]
                ov = out_v.at[b]

                @plsc.parallel_loop(0, CHUNK, step=16, unroll=8)
                def _(i):
                    flat = i + lane
                    row = lax.shift_right_logical(flat, 7)
                    col = lax.bitwise_and(flat, 127)
                    s_idx = sv[pl.ds(i, 16)]
                    d_idx = dv[pl.ds(i, 16)]
                    vs = plsc.load_gather(table_v, [s_idx])
                    vd = plsc.load_gather(table_v, [d_idx])
                    plsc.store_scatter(ov, [row, col], vs * vd)

                start_out(k, b)

                @pl.when(k + 2 < n_my)
                def _():
                    start_in(k + 2, b)

        return carry

    lax.fori_loop(0, MAX_PAIRS, pair_body, 0)
    wait_out(0)
    wait_out(1)


def kernel(node_attn_log_logits, edge_index):
    table = _node_sigmoid(node_attn_log_logits)
    out = _edge_attn_sc(table, edge_index.astype(jnp.int32))
    return out.reshape(N_EDGES, 1)
